# baseline (speedup 1.0000x reference)
import functools

import jax
import jax.numpy as jnp
from jax import lax
from jax.experimental import pallas as pl
from jax.experimental.pallas import tpu as pltpu
from jax.experimental.pallas import tpu_sc as plsc

DIM = 3
NUM_LEVELS = 16
NUM_FEATS = 2
BASE_R = 16
HASH_SIZE = 1 << 19
PRIME1_I32 = 2654435761 - (1 << 32)
PRIME2_I32 = 805459861
NUM_POINTS = 262144


def _level_offsets():
    offs = []
    off = 0
    for l in range(NUM_LEVELS):
        n = BASE_R << l
        t = min(HASH_SIZE, (n + 1) ** 3)
        offs.append(off)
        off += t
    offs.append(off)
    return offs


LEVEL_OFFS = _level_offsets()
TABLE_ROWS = LEVEL_OFFS[-1]

NUM_WORKERS = 32
PTS_PER_W = NUM_POINTS // NUM_WORKERS
CPTS = 128
NV = CPTS // 16
NCHUNK = PTS_PER_W // CPTS
GATH = CPTS * NUM_LEVELS * 8
OUT_COLS = NUM_LEVELS * NUM_FEATS

_mesh = plsc.VectorSubcoreMesh(core_axis_name="c", subcore_axis_name="s")


def _corner_terms(c0, l):
    if l < 3:
        res = BASE_R << l
        tx = (c0[0], c0[0] + 1)
        ty = (c0[1] * res, c0[1] * res + res)
        r2 = res * res
        tz = (c0[2] * r2, c0[2] * r2 + r2)
    else:
        tx = (c0[0], c0[0] + 1)
        y0 = c0[1] * PRIME1_I32
        z0 = c0[2] * PRIME2_I32
        ty = (y0, y0 + PRIME1_I32)
        tz = (z0, z0 + PRIME2_I32)
    return tx, ty, tz


def _corner_row(tx, ty, tz, corner, l):
    a = tx[corner & 1]
    b = ty[(corner >> 1) & 1]
    c = tz[(corner >> 2) & 1]
    if l < 3:
        return a + b + c + LEVEL_OFFS[l]
    return ((a ^ b ^ c) & (HASH_SIZE - 1)) + LEVEL_OFFS[l]


@functools.partial(
    pl.kernel,
    out_type=jax.ShapeDtypeStruct((NUM_POINTS * OUT_COLS,), jnp.float32),
    mesh=_mesh,
    scratch_types=[
        pltpu.VMEM((DIM, CPTS), jnp.float32),
        pltpu.VMEM((2 * GATH,), jnp.int32),
        pltpu.VMEM((2 * GATH,), jnp.float32),
        pltpu.VMEM((CPTS * OUT_COLS,), jnp.float32),
        pltpu.SemaphoreType.DMA,
    ],
)
def _encode(xt_hbm, tab_hbm, out_hbm, in_v, idx_v, rows_v, out_v, sem):
    wid = lax.axis_index("s") * 2 + lax.axis_index("c")
    iota = lax.iota(jnp.int32, 16)

    @pl.loop(0, NCHUNK)
    def _chunk(ci):
        base = wid * PTS_PER_W + ci * CPTS
        for d in range(DIM):
            pltpu.sync_copy(xt_hbm.at[pl.ds(d * NUM_POINTS + base, CPTS)], in_v.at[d])

        @pl.loop(0, NV)
        def _pass1(v):
            p0 = v * 16
            xs = [(in_v[d, pl.ds(p0, 16)] + 1.0) * 0.5 for d in range(DIM)]
            for l in range(NUM_LEVELS):
                scale = float(BASE_R * (2 ** l) - 1.0)
                c0 = [(xs[d] * scale + 0.5).astype(jnp.int32) for d in range(DIM)]
                tx, ty, tz = _corner_terms(c0, l)
                for corner in range(8):
                    g0 = (l * 8 + corner) * CPTS + p0
                    w0 = _corner_row(tx, ty, tz, corner, l) * 2
                    idx_v[pl.ds(g0, 16)] = w0
                    idx_v[pl.ds(GATH + g0, 16)] = w0 + 1

        pltpu.async_copy(tab_hbm.at[idx_v], rows_v, sem).wait()

        @pl.loop(0, NV)
        def _pass2(v):
            p0 = v * 16
            xs = [(in_v[d, pl.ds(p0, 16)] + 1.0) * 0.5 for d in range(DIM)]
            for l in range(NUM_LEVELS):
                scale = float(BASE_R * (2 ** l) - 1.0)
                fr = []
                for d in range(DIM):
                    pos = xs[d] * scale + 0.5
                    fr.append(pos - pos.astype(jnp.int32).astype(jnp.float32))
                wx = (1.0 - fr[0], fr[0])
                wy = (1.0 - fr[1], fr[1])
                wz = (1.0 - fr[2], fr[2])
                wyz = (wy[0] * wz[0], wy[1] * wz[0], wy[0] * wz[1], wy[1] * wz[1])
                acc0 = acc1 = None
                for corner in range(8):
                    g0 = (l * 8 + corner) * CPTS + p0
                    f0 = rows_v[pl.ds(g0, 16)]
                    f1 = rows_v[pl.ds(GATH + g0, 16)]
                    w = wx[corner & 1] * wyz[corner >> 1]
                    acc0 = w * f0 if acc0 is None else acc0 + w * f0
                    acc1 = w * f1 if acc1 is None else acc1 + w * f1
                out_v[pl.ds((2 * l) * CPTS + p0, 16)] = acc0
                out_v[pl.ds((2 * l + 1) * CPTS + p0, 16)] = acc1

        pltpu.sync_copy(out_v, out_hbm.at[pl.ds(base * OUT_COLS, CPTS * OUT_COLS)])


def kernel(inputs, embeddings):
    xt = inputs.T.reshape(-1)
    out = _encode(xt, embeddings.reshape(-1))
    return (
        out.reshape(NUM_POINTS // CPTS, OUT_COLS, CPTS)
        .transpose(0, 2, 1)
        .reshape(NUM_POINTS, OUT_COLS)
    )
